# baseline (speedup 1.0000x reference)
import functools
import jax
import jax.numpy as jnp
from jax.experimental import pallas as pl
from jax.experimental.pallas import tpu as pltpu

B = 2048
DIM = 2048
NUM_CODES = 8192
TOPK = 32
EPS = 1e-5


def _ln_body(x_ref, xn_ref, mu_ref, std_ref):
    x = x_ref[...]
    mu = jnp.mean(x, axis=1, keepdims=True)
    xc = x - mu
    var = jnp.sum(xc * xc, axis=1, keepdims=True) / (DIM - 1)
    std = jnp.sqrt(var)
    xn = xc / (std + EPS)
    xn_ref[...] = xn.astype(jnp.bfloat16)
    mu_ref[...] = mu
    std_ref[...] = std


def _layernorm(x):
    BR = 256
    return pl.pallas_call(
        _ln_body,
        grid=(B // BR,),
        in_specs=[pl.BlockSpec((BR, DIM), lambda i: (i, 0))],
        out_specs=[
            pl.BlockSpec((BR, DIM), lambda i: (i, 0)),
            pl.BlockSpec((BR, 1), lambda i: (i, 0)),
            pl.BlockSpec((BR, 1), lambda i: (i, 0)),
        ],
        out_shape=[
            jax.ShapeDtypeStruct((B, DIM), jnp.bfloat16),
            jax.ShapeDtypeStruct((B, 1), jnp.float32),
            jax.ShapeDtypeStruct((B, 1), jnp.float32),
        ],
    )(x)


def _enc_body(xn_ref, w_ref, lat_ref, inv_ref, wn_s, inv_s):
    i = pl.program_id(1)

    @pl.when(i == 0)
    def _():
        w = w_ref[...]
        sq = jnp.sum(w * w, axis=1, keepdims=True)
        inv = 1.0 / jnp.maximum(jnp.sqrt(sq), 1e-12)
        inv_s[...] = inv
        wn_s[...] = (w * inv).astype(jnp.bfloat16)

    lat = jax.lax.dot_general(
        xn_ref[...], wn_s[...],
        (((1,), (1,)), ((), ())),
        preferred_element_type=jnp.float32,
    )
    lat_ref[...] = lat
    inv_ref[...] = inv_s[...]


def _encode(xn, W):
    BR, CB = 256, 1024
    return pl.pallas_call(
        _enc_body,
        grid=(NUM_CODES // CB, B // BR),
        in_specs=[
            pl.BlockSpec((BR, DIM), lambda j, i: (i, 0)),
            pl.BlockSpec((CB, DIM), lambda j, i: (j, 0)),
        ],
        out_specs=[
            pl.BlockSpec((BR, CB), lambda j, i: (i, j)),
            pl.BlockSpec((CB, 1), lambda j, i: (j, 0)),
        ],
        out_shape=[
            jax.ShapeDtypeStruct((B, NUM_CODES), jnp.float32),
            jax.ShapeDtypeStruct((NUM_CODES, 1), jnp.float32),
        ],
        scratch_shapes=[
            pltpu.VMEM((CB, DIM), jnp.bfloat16),
            pltpu.VMEM((CB, 1), jnp.float32),
        ],
    )(xn, W)


_NPLANE = 64
_TLEV = 16


def _bitonic_sort_desc(a):
    n = len(a)
    k = 2
    while k <= n:
        jj = k // 2
        while jj >= 1:
            for i in range(n):
                l = i ^ jj
                if l > i:
                    hi = jnp.maximum(a[i], a[l])
                    lo = jnp.minimum(a[i], a[l])
                    if (i & k) == 0:
                        a[i], a[l] = hi, lo
                    else:
                        a[i], a[l] = lo, hi
            jj //= 2
        k *= 2
    return a


def _top16_merge(A, Bl):
    c = [jnp.maximum(A[i], Bl[15 - i]) for i in range(16)]
    for jj in (8, 4, 2, 1):
        for i in range(16):
            l = i ^ jj
            if l > i:
                hi = jnp.maximum(c[i], c[l])
                lo = jnp.minimum(c[i], c[l])
                c[i], c[l] = hi, lo
    return c


_QROWS = 8


def _thresh_body(lat_ref, thr_ref, masked_ref):
    br = thr_ref.shape[0]
    cols = [lat_ref[:, 128 * j:128 * (j + 1)] for j in range(_NPLANE)]
    groups = [
        _bitonic_sort_desc(cols[16 * g:16 * (g + 1)]) for g in range(4)
    ]
    m01 = _top16_merge(groups[0], groups[1])
    m23 = _top16_merge(groups[2], groups[3])
    S = _top16_merge(m01, m23)

    nq = br // _QROWS
    neg = jnp.full((_QROWS, 128), -jnp.inf, jnp.float32)
    Sq = [[p[q * _QROWS:(q + 1) * _QROWS, :] for p in S] for q in range(nq)]
    heads = [Sq[q][0] for q in range(nq)]
    d = [jnp.zeros((_QROWS, 128), jnp.int32) for _ in range(nq)]
    m = [None] * nq
    for _ in range(TOPK):
        for q in range(nq):
            m[q] = jnp.max(heads[q], axis=1, keepdims=True)
            hit = heads[q] == m[q]
            dq = d[q] + hit.astype(jnp.int32)
            d[q] = dq
            b0 = (dq & 1) > 0
            b1 = (dq & 2) > 0
            b2 = (dq & 4) > 0
            b3 = (dq & 8) > 0
            Sv = Sq[q]
            t0 = [jnp.where(b0, Sv[2 * i + 1], Sv[2 * i]) for i in range(8)]
            t1 = [jnp.where(b1, t0[2 * i + 1], t0[2 * i]) for i in range(4)]
            t2 = [jnp.where(b2, t1[2 * i + 1], t1[2 * i]) for i in range(2)]
            t3 = jnp.where(b3, t2[1], t2[0])
            nxt = jnp.where(dq >= _TLEV, neg, t3)
            heads[q] = jnp.where(hit, nxt, heads[q])
    thr = jnp.concatenate(m, axis=0)
    thr_ref[...] = thr
    lat = lat_ref[...]
    masked_ref[...] = jnp.where(
        lat >= thr, lat, 0.0
    ).astype(jnp.bfloat16)


def _thresholds(latents):
    BR = 64
    return pl.pallas_call(
        _thresh_body,
        grid=(B // BR,),
        in_specs=[pl.BlockSpec((BR, NUM_CODES), lambda i: (i, 0))],
        out_specs=[
            pl.BlockSpec((BR, 1), lambda i: (i, 0)),
            pl.BlockSpec((BR, NUM_CODES), lambda i: (i, 0)),
        ],
        out_shape=[
            jax.ShapeDtypeStruct((B, 1), jnp.float32),
            jax.ShapeDtypeStruct((B, NUM_CODES), jnp.bfloat16),
        ],
    )(latents)


def _dec_body(masked_ref, inv_ref, w_ref, mu_ref, std_ref, out_ref, wn_s):
    i = pl.program_id(1)

    @pl.when(i == 0)
    def _():
        wn_s[...] = (w_ref[...] * inv_ref[...]).astype(jnp.bfloat16)

    ret = jax.lax.dot_general(
        masked_ref[...], wn_s[...],
        (((1,), (0,)), ((), ())),
        preferred_element_type=jnp.float32,
    )
    out_ref[...] = ret * std_ref[...] + mu_ref[...]


def _decode(masked, inv, W, mu, std):
    BR, DB = 256, 512
    return pl.pallas_call(
        _dec_body,
        grid=(DIM // DB, B // BR),
        in_specs=[
            pl.BlockSpec((BR, NUM_CODES), lambda j, i: (i, 0)),
            pl.BlockSpec((NUM_CODES, 1), lambda j, i: (0, 0)),
            pl.BlockSpec((NUM_CODES, DB), lambda j, i: (0, j)),
            pl.BlockSpec((BR, 1), lambda j, i: (i, 0)),
            pl.BlockSpec((BR, 1), lambda j, i: (i, 0)),
        ],
        out_specs=pl.BlockSpec((BR, DB), lambda j, i: (i, j)),
        out_shape=jax.ShapeDtypeStruct((B, DIM), jnp.float32),
        scratch_shapes=[pltpu.VMEM((NUM_CODES, DB), jnp.bfloat16)],
    )(masked, inv, W, mu, std)


@jax.jit
def kernel(x, W):
    xn, mu, std = _layernorm(x)
    latents, inv = _encode(xn, W)
    thr, masked = _thresholds(latents)
    x_hat = _decode(masked, inv, W, mu, std)
    return (x_hat, latents)


# speedup vs baseline: 1.0557x; 1.0557x over previous
import functools
import jax
import jax.numpy as jnp
from jax.experimental import pallas as pl
from jax.experimental.pallas import tpu as pltpu

B = 2048
DIM = 2048
NUM_CODES = 8192
TOPK = 32
EPS = 1e-5


def _ln_body(x_ref, xn_ref, mu_ref, std_ref):
    x = x_ref[...]
    mu = jnp.mean(x, axis=1, keepdims=True)
    xc = x - mu
    var = jnp.sum(xc * xc, axis=1, keepdims=True) / (DIM - 1)
    std = jnp.sqrt(var)
    xn = xc / (std + EPS)
    xn_ref[...] = xn.astype(jnp.bfloat16)
    mu_ref[...] = mu
    std_ref[...] = std


def _layernorm(x):
    BR = 256
    return pl.pallas_call(
        _ln_body,
        grid=(B // BR,),
        in_specs=[pl.BlockSpec((BR, DIM), lambda i: (i, 0))],
        out_specs=[
            pl.BlockSpec((BR, DIM), lambda i: (i, 0)),
            pl.BlockSpec((BR, 1), lambda i: (i, 0)),
            pl.BlockSpec((BR, 1), lambda i: (i, 0)),
        ],
        out_shape=[
            jax.ShapeDtypeStruct((B, DIM), jnp.bfloat16),
            jax.ShapeDtypeStruct((B, 1), jnp.float32),
            jax.ShapeDtypeStruct((B, 1), jnp.float32),
        ],
    )(x)


def _enc_body(xn_ref, w_ref, lat_ref, inv_ref, wn_s, inv_s):
    i = pl.program_id(1)

    @pl.when(i == 0)
    def _():
        w = w_ref[...]
        sq = jnp.sum(w * w, axis=1, keepdims=True)
        inv = 1.0 / jnp.maximum(jnp.sqrt(sq), 1e-12)
        inv_s[...] = inv
        wn_s[...] = (w * inv).astype(jnp.bfloat16)

    lat = jax.lax.dot_general(
        xn_ref[...], wn_s[...],
        (((1,), (1,)), ((), ())),
        preferred_element_type=jnp.float32,
    )
    lat_ref[...] = lat
    inv_ref[...] = inv_s[...]


def _encode(xn, W):
    BR, CB = 256, 2048
    return pl.pallas_call(
        _enc_body,
        grid=(NUM_CODES // CB, B // BR),
        in_specs=[
            pl.BlockSpec((BR, DIM), lambda j, i: (i, 0)),
            pl.BlockSpec((CB, DIM), lambda j, i: (j, 0)),
        ],
        out_specs=[
            pl.BlockSpec((BR, CB), lambda j, i: (i, j)),
            pl.BlockSpec((CB, 1), lambda j, i: (j, 0)),
        ],
        out_shape=[
            jax.ShapeDtypeStruct((B, NUM_CODES), jnp.float32),
            jax.ShapeDtypeStruct((NUM_CODES, 1), jnp.float32),
        ],
        scratch_shapes=[
            pltpu.VMEM((CB, DIM), jnp.bfloat16),
            pltpu.VMEM((CB, 1), jnp.float32),
        ],
    )(xn, W)


_NPLANE = 64
_TLEV = 16


def _bitonic_sort_desc(a):
    n = len(a)
    k = 2
    while k <= n:
        jj = k // 2
        while jj >= 1:
            for i in range(n):
                l = i ^ jj
                if l > i:
                    hi = jnp.maximum(a[i], a[l])
                    lo = jnp.minimum(a[i], a[l])
                    if (i & k) == 0:
                        a[i], a[l] = hi, lo
                    else:
                        a[i], a[l] = lo, hi
            jj //= 2
        k *= 2
    return a


def _top16_merge(A, Bl):
    c = [jnp.maximum(A[i], Bl[15 - i]) for i in range(16)]
    for jj in (8, 4, 2, 1):
        for i in range(16):
            l = i ^ jj
            if l > i:
                hi = jnp.maximum(c[i], c[l])
                lo = jnp.minimum(c[i], c[l])
                c[i], c[l] = hi, lo
    return c


_QROWS = 8


def _thresh_body(lat_ref, thr_ref, masked_ref):
    br = thr_ref.shape[0]
    cols = [lat_ref[:, 128 * j:128 * (j + 1)] for j in range(_NPLANE)]
    groups = [
        _bitonic_sort_desc(cols[16 * g:16 * (g + 1)]) for g in range(4)
    ]
    m01 = _top16_merge(groups[0], groups[1])
    m23 = _top16_merge(groups[2], groups[3])
    S = _top16_merge(m01, m23)

    nq = br // _QROWS
    neg = jnp.full((_QROWS, 128), -jnp.inf, jnp.float32)
    Sq = [[p[q * _QROWS:(q + 1) * _QROWS, :] for p in S] for q in range(nq)]
    heads = [Sq[q][0] for q in range(nq)]
    d = [jnp.zeros((_QROWS, 128), jnp.int32) for _ in range(nq)]
    m = [None] * nq
    for _ in range(TOPK):
        for q in range(nq):
            m[q] = jnp.max(heads[q], axis=1, keepdims=True)
            hit = heads[q] == m[q]
            dq = d[q] + hit.astype(jnp.int32)
            d[q] = dq
            b0 = (dq & 1) > 0
            b1 = (dq & 2) > 0
            b2 = (dq & 4) > 0
            b3 = (dq & 8) > 0
            Sv = Sq[q]
            t0 = [jnp.where(b0, Sv[2 * i + 1], Sv[2 * i]) for i in range(8)]
            t1 = [jnp.where(b1, t0[2 * i + 1], t0[2 * i]) for i in range(4)]
            t2 = [jnp.where(b2, t1[2 * i + 1], t1[2 * i]) for i in range(2)]
            t3 = jnp.where(b3, t2[1], t2[0])
            nxt = jnp.where(dq >= _TLEV, neg, t3)
            heads[q] = jnp.where(hit, nxt, heads[q])
    thr = jnp.concatenate(m, axis=0)
    thr_ref[...] = thr
    lat = lat_ref[...]
    masked_ref[...] = jnp.where(
        lat >= thr, lat, 0.0
    ).astype(jnp.bfloat16)


def _thresholds(latents):
    BR = 64
    return pl.pallas_call(
        _thresh_body,
        grid=(B // BR,),
        in_specs=[pl.BlockSpec((BR, NUM_CODES), lambda i: (i, 0))],
        out_specs=[
            pl.BlockSpec((BR, 1), lambda i: (i, 0)),
            pl.BlockSpec((BR, NUM_CODES), lambda i: (i, 0)),
        ],
        out_shape=[
            jax.ShapeDtypeStruct((B, 1), jnp.float32),
            jax.ShapeDtypeStruct((B, NUM_CODES), jnp.bfloat16),
        ],
    )(latents)


def _dec_body(masked_ref, inv_ref, w_ref, mu_ref, std_ref, out_ref, wn_s):
    i = pl.program_id(1)

    @pl.when(i == 0)
    def _():
        wn_s[...] = (w_ref[...] * inv_ref[...]).astype(jnp.bfloat16)

    ret = jax.lax.dot_general(
        masked_ref[...], wn_s[...],
        (((1,), (0,)), ((), ())),
        preferred_element_type=jnp.float32,
    )
    out_ref[...] = ret * std_ref[...] + mu_ref[...]


def _decode(masked, inv, W, mu, std):
    BR, DB = 256, 512
    return pl.pallas_call(
        _dec_body,
        grid=(DIM // DB, B // BR),
        in_specs=[
            pl.BlockSpec((BR, NUM_CODES), lambda j, i: (i, 0)),
            pl.BlockSpec((NUM_CODES, 1), lambda j, i: (0, 0)),
            pl.BlockSpec((NUM_CODES, DB), lambda j, i: (0, j)),
            pl.BlockSpec((BR, 1), lambda j, i: (i, 0)),
            pl.BlockSpec((BR, 1), lambda j, i: (i, 0)),
        ],
        out_specs=pl.BlockSpec((BR, DB), lambda j, i: (i, j)),
        out_shape=jax.ShapeDtypeStruct((B, DIM), jnp.float32),
        scratch_shapes=[pltpu.VMEM((NUM_CODES, DB), jnp.bfloat16)],
    )(masked, inv, W, mu, std)


@jax.jit
def kernel(x, W):
    xn, mu, std = _layernorm(x)
    latents, inv = _encode(xn, W)
    thr, masked = _thresholds(latents)
    x_hat = _decode(masked, inv, W, mu, std)
    return (x_hat, latents)


# speedup vs baseline: 1.0856x; 1.0284x over previous
import functools
import jax
import jax.numpy as jnp
from jax.experimental import pallas as pl
from jax.experimental.pallas import tpu as pltpu

B = 2048
DIM = 2048
NUM_CODES = 8192
TOPK = 32
EPS = 1e-5


def _ln_body(x_ref, xn_ref, mu_ref, std_ref):
    x = x_ref[...]
    mu = jnp.mean(x, axis=1, keepdims=True)
    xc = x - mu
    var = jnp.sum(xc * xc, axis=1, keepdims=True) / (DIM - 1)
    std = jnp.sqrt(var)
    xn = xc / (std + EPS)
    xn_ref[...] = xn.astype(jnp.bfloat16)
    mu_ref[...] = mu
    std_ref[...] = std


def _layernorm(x):
    BR = 256
    return pl.pallas_call(
        _ln_body,
        grid=(B // BR,),
        in_specs=[pl.BlockSpec((BR, DIM), lambda i: (i, 0))],
        out_specs=[
            pl.BlockSpec((BR, DIM), lambda i: (i, 0)),
            pl.BlockSpec((BR, 1), lambda i: (i, 0)),
            pl.BlockSpec((BR, 1), lambda i: (i, 0)),
        ],
        out_shape=[
            jax.ShapeDtypeStruct((B, DIM), jnp.bfloat16),
            jax.ShapeDtypeStruct((B, 1), jnp.float32),
            jax.ShapeDtypeStruct((B, 1), jnp.float32),
        ],
    )(x)


def _enc_body(xn_ref, w_ref, lat_ref, inv_ref, wn_s, inv_s):
    i = pl.program_id(1)

    @pl.when(i == 0)
    def _():
        w = w_ref[...]
        sq = jnp.sum(w * w, axis=1, keepdims=True)
        inv = 1.0 / jnp.maximum(jnp.sqrt(sq), 1e-12)
        inv_s[...] = inv
        wn_s[...] = (w * inv).astype(jnp.bfloat16)

    lat = jax.lax.dot_general(
        xn_ref[...], wn_s[...],
        (((1,), (1,)), ((), ())),
        preferred_element_type=jnp.float32,
    )
    lat_ref[...] = lat
    inv_ref[...] = inv_s[...]


def _encode(xn, W):
    BR, CB = 256, 2048
    return pl.pallas_call(
        _enc_body,
        grid=(NUM_CODES // CB, B // BR),
        in_specs=[
            pl.BlockSpec((BR, DIM), lambda j, i: (i, 0)),
            pl.BlockSpec((CB, DIM), lambda j, i: (j, 0)),
        ],
        out_specs=[
            pl.BlockSpec((BR, CB), lambda j, i: (i, j)),
            pl.BlockSpec((CB, 1), lambda j, i: (j, 0)),
        ],
        out_shape=[
            jax.ShapeDtypeStruct((B, NUM_CODES), jnp.float32),
            jax.ShapeDtypeStruct((NUM_CODES, 1), jnp.float32),
        ],
        scratch_shapes=[
            pltpu.VMEM((CB, DIM), jnp.bfloat16),
            pltpu.VMEM((CB, 1), jnp.float32),
        ],
    )(xn, W)


_NPLANE = 64
_TLEV = 8


def _bitonic_sort_desc(a):
    n = len(a)
    k = 2
    while k <= n:
        jj = k // 2
        while jj >= 1:
            for i in range(n):
                l = i ^ jj
                if l > i:
                    hi = jnp.maximum(a[i], a[l])
                    lo = jnp.minimum(a[i], a[l])
                    if (i & k) == 0:
                        a[i], a[l] = hi, lo
                    else:
                        a[i], a[l] = lo, hi
            jj //= 2
        k *= 2
    return a


def _topT_merge(A, Bl):
    T = len(A)
    c = [jnp.maximum(A[i], Bl[T - 1 - i]) for i in range(T)]
    jj = T // 2
    while jj >= 1:
        for i in range(T):
            l = i ^ jj
            if l > i:
                hi = jnp.maximum(c[i], c[l])
                lo = jnp.minimum(c[i], c[l])
                c[i], c[l] = hi, lo
        jj //= 2
    return c


_QROWS = 8


def _thresh_body(lat_ref, thr_ref, masked_ref):
    br = thr_ref.shape[0]
    cols = [lat_ref[:, 128 * j:128 * (j + 1)] for j in range(_NPLANE)]
    lists = [
        _bitonic_sort_desc(cols[8 * g:8 * (g + 1)]) for g in range(8)
    ]
    while len(lists) > 1:
        lists = [
            _topT_merge(lists[2 * g], lists[2 * g + 1])
            for g in range(len(lists) // 2)
        ]
    S = lists[0]

    nq = br // _QROWS
    neg = jnp.full((_QROWS, 128), -jnp.inf, jnp.float32)
    Sq = [[p[q * _QROWS:(q + 1) * _QROWS, :] for p in S] for q in range(nq)]
    heads = [Sq[q][0] for q in range(nq)]
    d = [jnp.zeros((_QROWS, 128), jnp.int32) for _ in range(nq)]
    m = [None] * nq
    for _ in range(TOPK):
        for q in range(nq):
            m[q] = jnp.max(heads[q], axis=1, keepdims=True)
            hit = heads[q] == m[q]
            dq = d[q] + hit.astype(jnp.int32)
            d[q] = dq
            b0 = (dq & 1) > 0
            b1 = (dq & 2) > 0
            b2 = (dq & 4) > 0
            Sv = Sq[q]
            t0 = [jnp.where(b0, Sv[2 * i + 1], Sv[2 * i]) for i in range(4)]
            t1 = [jnp.where(b1, t0[2 * i + 1], t0[2 * i]) for i in range(2)]
            t2 = jnp.where(b2, t1[1], t1[0])
            nxt = jnp.where(dq >= _TLEV, neg, t2)
            heads[q] = jnp.where(hit, nxt, heads[q])
    thr = jnp.concatenate(m, axis=0)
    thr_ref[...] = thr
    lat = lat_ref[...]
    masked_ref[...] = jnp.where(
        lat >= thr, lat, 0.0
    ).astype(jnp.bfloat16)


def _thresholds(latents):
    BR = 64
    return pl.pallas_call(
        _thresh_body,
        grid=(B // BR,),
        in_specs=[pl.BlockSpec((BR, NUM_CODES), lambda i: (i, 0))],
        out_specs=[
            pl.BlockSpec((BR, 1), lambda i: (i, 0)),
            pl.BlockSpec((BR, NUM_CODES), lambda i: (i, 0)),
        ],
        out_shape=[
            jax.ShapeDtypeStruct((B, 1), jnp.float32),
            jax.ShapeDtypeStruct((B, NUM_CODES), jnp.bfloat16),
        ],
    )(latents)


def _dec_body(masked_ref, inv_ref, w_ref, mu_ref, std_ref, out_ref, wn_s):
    i = pl.program_id(1)

    @pl.when(i == 0)
    def _():
        wn_s[...] = (w_ref[...] * inv_ref[...]).astype(jnp.bfloat16)

    ret = jax.lax.dot_general(
        masked_ref[...], wn_s[...],
        (((1,), (0,)), ((), ())),
        preferred_element_type=jnp.float32,
    )
    out_ref[...] = ret * std_ref[...] + mu_ref[...]


def _decode(masked, inv, W, mu, std):
    BR, DB = 256, 512
    return pl.pallas_call(
        _dec_body,
        grid=(DIM // DB, B // BR),
        in_specs=[
            pl.BlockSpec((BR, NUM_CODES), lambda j, i: (i, 0)),
            pl.BlockSpec((NUM_CODES, 1), lambda j, i: (0, 0)),
            pl.BlockSpec((NUM_CODES, DB), lambda j, i: (0, j)),
            pl.BlockSpec((BR, 1), lambda j, i: (i, 0)),
            pl.BlockSpec((BR, 1), lambda j, i: (i, 0)),
        ],
        out_specs=pl.BlockSpec((BR, DB), lambda j, i: (i, j)),
        out_shape=jax.ShapeDtypeStruct((B, DIM), jnp.float32),
        scratch_shapes=[pltpu.VMEM((NUM_CODES, DB), jnp.bfloat16)],
    )(masked, inv, W, mu, std)


@jax.jit
def kernel(x, W):
    xn, mu, std = _layernorm(x)
    latents, inv = _encode(xn, W)
    thr, masked = _thresholds(latents)
    x_hat = _decode(masked, inv, W, mu, std)
    return (x_hat, latents)


# speedup vs baseline: 1.3349x; 1.2296x over previous
import functools
import jax
import jax.numpy as jnp
from jax.experimental import pallas as pl
from jax.experimental.pallas import tpu as pltpu

B = 2048
DIM = 2048
NUM_CODES = 8192
TOPK = 32
EPS = 1e-5


def _ln_body(x_ref, xn_ref, mu_ref, std_ref):
    x = x_ref[...]
    mu = jnp.mean(x, axis=1, keepdims=True)
    xc = x - mu
    var = jnp.sum(xc * xc, axis=1, keepdims=True) / (DIM - 1)
    std = jnp.sqrt(var)
    xn = xc / (std + EPS)
    xn_ref[...] = xn.astype(jnp.bfloat16)
    mu_ref[...] = mu
    std_ref[...] = std


def _layernorm(x):
    BR = 256
    return pl.pallas_call(
        _ln_body,
        grid=(B // BR,),
        in_specs=[pl.BlockSpec((BR, DIM), lambda i: (i, 0))],
        out_specs=[
            pl.BlockSpec((BR, DIM), lambda i: (i, 0)),
            pl.BlockSpec((BR, 1), lambda i: (i, 0)),
            pl.BlockSpec((BR, 1), lambda i: (i, 0)),
        ],
        out_shape=[
            jax.ShapeDtypeStruct((B, DIM), jnp.bfloat16),
            jax.ShapeDtypeStruct((B, 1), jnp.float32),
            jax.ShapeDtypeStruct((B, 1), jnp.float32),
        ],
    )(x)


def _enc_body(xn_ref, w_ref, lat_ref, wn_ref):
    i = pl.program_id(1)

    @pl.when(i == 0)
    def _():
        w = w_ref[...]
        sq = jnp.sum(w * w, axis=1, keepdims=True)
        inv = 1.0 / jnp.maximum(jnp.sqrt(sq), 1e-12)
        wn_ref[...] = (w * inv).astype(jnp.bfloat16)

    lat = jax.lax.dot_general(
        xn_ref[...], wn_ref[...],
        (((1,), (1,)), ((), ())),
        preferred_element_type=jnp.float32,
    )
    lat_ref[...] = lat


def _encode(xn, W):
    BR, CB = 256, 2048
    return pl.pallas_call(
        _enc_body,
        grid=(NUM_CODES // CB, B // BR),
        in_specs=[
            pl.BlockSpec((BR, DIM), lambda j, i: (i, 0)),
            pl.BlockSpec((CB, DIM), lambda j, i: (j, 0)),
        ],
        out_specs=[
            pl.BlockSpec((BR, CB), lambda j, i: (i, j)),
            pl.BlockSpec((CB, DIM), lambda j, i: (j, 0)),
        ],
        out_shape=[
            jax.ShapeDtypeStruct((B, NUM_CODES), jnp.float32),
            jax.ShapeDtypeStruct((NUM_CODES, DIM), jnp.bfloat16),
        ],
    )(xn, W)


_NPLANE = 64
_TLEV = 8


def _bitonic_sort_desc(a):
    n = len(a)
    k = 2
    while k <= n:
        jj = k // 2
        while jj >= 1:
            for i in range(n):
                l = i ^ jj
                if l > i:
                    hi = jnp.maximum(a[i], a[l])
                    lo = jnp.minimum(a[i], a[l])
                    if (i & k) == 0:
                        a[i], a[l] = hi, lo
                    else:
                        a[i], a[l] = lo, hi
            jj //= 2
        k *= 2
    return a


def _topT_merge(A, Bl):
    T = len(A)
    c = [jnp.maximum(A[i], Bl[T - 1 - i]) for i in range(T)]
    jj = T // 2
    while jj >= 1:
        for i in range(T):
            l = i ^ jj
            if l > i:
                hi = jnp.maximum(c[i], c[l])
                lo = jnp.minimum(c[i], c[l])
                c[i], c[l] = hi, lo
        jj //= 2
    return c


_QROWS = 8


def _seldec_body(lat_ref, wn_ref, mu_ref, std_ref, out_ref):
    br = mu_ref.shape[0]
    cols = [lat_ref[:, 128 * j:128 * (j + 1)] for j in range(_NPLANE)]
    lists = [
        _bitonic_sort_desc(cols[8 * g:8 * (g + 1)]) for g in range(8)
    ]
    while len(lists) > 1:
        lists = [
            _topT_merge(lists[2 * g], lists[2 * g + 1])
            for g in range(len(lists) // 2)
        ]
    S = lists[0]

    nq = br // _QROWS
    neg = jnp.full((_QROWS, 128), -jnp.inf, jnp.float32)
    Sq = [[p[q * _QROWS:(q + 1) * _QROWS, :] for p in S] for q in range(nq)]
    heads = [Sq[q][0] for q in range(nq)]
    d = [jnp.zeros((_QROWS, 128), jnp.int32) for _ in range(nq)]
    m = [None] * nq
    for _ in range(TOPK):
        for q in range(nq):
            m[q] = jnp.max(heads[q], axis=1, keepdims=True)
            hit = heads[q] == m[q]
            dq = d[q] + hit.astype(jnp.int32)
            d[q] = dq
            b0 = (dq & 1) > 0
            b1 = (dq & 2) > 0
            b2 = (dq & 4) > 0
            Sv = Sq[q]
            t0 = [jnp.where(b0, Sv[2 * i + 1], Sv[2 * i]) for i in range(4)]
            t1 = [jnp.where(b1, t0[2 * i + 1], t0[2 * i]) for i in range(2)]
            t2 = jnp.where(b2, t1[1], t1[0])
            nxt = jnp.where(dq >= _TLEV, neg, t2)
            heads[q] = jnp.where(hit, nxt, heads[q])
    thr = jnp.concatenate(m, axis=0)
    lat = lat_ref[...]
    masked = jnp.where(lat >= thr, lat, 0.0).astype(jnp.bfloat16)
    ret = jax.lax.dot_general(
        masked, wn_ref[...],
        (((1,), (0,)), ((), ())),
        preferred_element_type=jnp.float32,
    )
    out_ref[...] = ret * std_ref[...] + mu_ref[...]


def _select_decode(latents, wn, mu, std):
    BR = 128
    return pl.pallas_call(
        _seldec_body,
        grid=(B // BR,),
        in_specs=[
            pl.BlockSpec((BR, NUM_CODES), lambda i: (i, 0)),
            pl.BlockSpec((NUM_CODES, DIM), lambda i: (0, 0)),
            pl.BlockSpec((BR, 1), lambda i: (i, 0)),
            pl.BlockSpec((BR, 1), lambda i: (i, 0)),
        ],
        out_specs=pl.BlockSpec((BR, DIM), lambda i: (i, 0)),
        out_shape=jax.ShapeDtypeStruct((B, DIM), jnp.float32),
    )(latents, wn, mu, std)


@jax.jit
def kernel(x, W):
    xn, mu, std = _layernorm(x)
    latents, wn = _encode(xn, W)
    x_hat = _select_decode(latents, wn, mu, std)
    return (x_hat, latents)
